# Initial kernel scaffold; baseline (speedup 1.0000x reference)
import jax
import jax.numpy as jnp
from jax.experimental import pallas as pl


def kernel(x, gW, gb, W1, b1, W2, b2):
    raise NotImplementedError("write your pallas kernel here")


# speedup vs baseline: 5.1921x; 5.1921x over previous
import functools

import jax
import jax.numpy as jnp
from jax.experimental import pallas as pl
from jax.experimental.pallas import tpu as pltpu

E = 8
TOP_K = 2
D = 1024
T = 1024
H = 1024
B = 2048

GATE_BB = 256
FFN_BB = 512


def _gating_body(x_ref, gw_ref, vals_ref, idx_ref, w_ref):
    gw = gw_ref[0, :].astype(jnp.bfloat16).astype(jnp.float32)
    logits = []
    for e in range(E):
        xe = x_ref[e].astype(jnp.bfloat16).astype(jnp.float32)
        logits.append(jnp.sum(xe * gw[None, :], axis=1, keepdims=True))
    lg = jnp.concatenate(logits, axis=1)
    m = jnp.max(lg, axis=1, keepdims=True)
    ex = jnp.exp(lg - m)
    p = ex / jnp.sum(ex, axis=1, keepdims=True)

    lane = jax.lax.broadcasted_iota(jnp.int32, p.shape, 1)
    m1 = jnp.max(p, axis=1, keepdims=True)
    i1 = jnp.min(jnp.where(p == m1, lane, E), axis=1, keepdims=True)
    p2 = jnp.where(lane == i1, -jnp.inf, p)
    m2 = jnp.max(p2, axis=1, keepdims=True)
    i2 = jnp.min(jnp.where(p2 == m2, lane, E), axis=1, keepdims=True)

    vals_ref[...] = jnp.concatenate([m1, m2], axis=1)
    idx_ref[...] = jnp.concatenate([i1, i2], axis=1)
    w_ref[...] = jnp.where(lane == i1, m1, 0.0) + jnp.where(lane == i2, m2, 0.0)


def _ffn_body(x_ref, w1_ref, b1_ref, w2_ref, b2_ref, wt_ref, out_ref):
    e = pl.program_id(1)
    xe = x_ref[0]
    h = jnp.maximum(
        jnp.dot(xe, w1_ref[0], preferred_element_type=jnp.float32) + b1_ref[0], 0.0
    )
    o = jnp.dot(h, w2_ref[0], preferred_element_type=jnp.float32) + b2_ref[0]
    lane = jax.lax.broadcasted_iota(jnp.int32, wt_ref.shape, 1)
    wcol = jnp.sum(jnp.where(lane == e, wt_ref[...], 0.0), axis=1, keepdims=True)
    contrib = wcol * o

    @pl.when(e == 0)
    def _init():
        out_ref[...] = contrib

    @pl.when(e != 0)
    def _acc():
        out_ref[...] += contrib


@jax.jit
def kernel(x, gW, gb, W1, b1, W2, b2):
    del gb
    gw2 = gW.reshape(1, D)

    vals, idx, wts = pl.pallas_call(
        _gating_body,
        grid=(B // GATE_BB,),
        in_specs=[
            pl.BlockSpec((E, GATE_BB, D), lambda i: (0, i, 0)),
            pl.BlockSpec((1, D), lambda i: (0, 0)),
        ],
        out_specs=[
            pl.BlockSpec((GATE_BB, TOP_K), lambda i: (i, 0)),
            pl.BlockSpec((GATE_BB, TOP_K), lambda i: (i, 0)),
            pl.BlockSpec((GATE_BB, E), lambda i: (i, 0)),
        ],
        out_shape=[
            jax.ShapeDtypeStruct((B, TOP_K), jnp.float32),
            jax.ShapeDtypeStruct((B, TOP_K), jnp.int32),
            jax.ShapeDtypeStruct((B, E), jnp.float32),
        ],
    )(x, gw2)

    b1r = b1.reshape(E, 1, H)
    b2r = b2.reshape(E, 1, T)
    out = pl.pallas_call(
        _ffn_body,
        grid=(B // FFN_BB, E),
        in_specs=[
            pl.BlockSpec((1, FFN_BB, D), lambda i, e: (e, i, 0)),
            pl.BlockSpec((1, D, H), lambda i, e: (e, 0, 0)),
            pl.BlockSpec((1, 1, H), lambda i, e: (e, 0, 0)),
            pl.BlockSpec((1, H, T), lambda i, e: (e, 0, 0)),
            pl.BlockSpec((1, 1, T), lambda i, e: (e, 0, 0)),
            pl.BlockSpec((FFN_BB, E), lambda i, e: (i, 0)),
        ],
        out_specs=pl.BlockSpec((FFN_BB, T), lambda i, e: (i, 0)),
        out_shape=jax.ShapeDtypeStruct((B, T), jnp.float32),
        compiler_params=pltpu.CompilerParams(
            dimension_semantics=("arbitrary", "arbitrary"),
        ),
    )(x, W1, b1r, W2, b2r, wts)

    return (out, vals)


# speedup vs baseline: 6.2088x; 1.1958x over previous
import functools

import jax
import jax.numpy as jnp
from jax.experimental import pallas as pl
from jax.experimental.pallas import tpu as pltpu

E = 8
TOP_K = 2
D = 1024
T = 1024
H = 1024
B = 2048

GATE_BB = 256
FFN_BB = 2048


def _gating_body(x_ref, gw_ref, vals_ref, idx_ref, w_ref):
    gw = gw_ref[0, :].astype(jnp.bfloat16).astype(jnp.float32)
    logits = []
    for e in range(E):
        xe = x_ref[e].astype(jnp.bfloat16).astype(jnp.float32)
        logits.append(jnp.sum(xe * gw[None, :], axis=1, keepdims=True))
    lg = jnp.concatenate(logits, axis=1)
    m = jnp.max(lg, axis=1, keepdims=True)
    ex = jnp.exp(lg - m)
    p = ex / jnp.sum(ex, axis=1, keepdims=True)

    lane = jax.lax.broadcasted_iota(jnp.int32, p.shape, 1)
    m1 = jnp.max(p, axis=1, keepdims=True)
    i1 = jnp.min(jnp.where(p == m1, lane, E), axis=1, keepdims=True)
    p2 = jnp.where(lane == i1, -jnp.inf, p)
    m2 = jnp.max(p2, axis=1, keepdims=True)
    i2 = jnp.min(jnp.where(p2 == m2, lane, E), axis=1, keepdims=True)

    vals_ref[...] = jnp.concatenate([m1, m2], axis=1)
    idx_ref[...] = jnp.concatenate([i1, i2], axis=1)
    w_ref[...] = jnp.where(lane == i1, m1, 0.0) + jnp.where(lane == i2, m2, 0.0)


def _ffn_body(x_ref, w1_ref, b1_ref, w2_ref, b2_ref, wt_ref, out_ref):
    e = pl.program_id(1)
    xe = x_ref[0]
    h = jnp.maximum(
        jnp.dot(xe, w1_ref[0], preferred_element_type=jnp.float32) + b1_ref[0], 0.0
    )
    o = jnp.dot(h, w2_ref[0], preferred_element_type=jnp.float32) + b2_ref[0]
    lane = jax.lax.broadcasted_iota(jnp.int32, wt_ref.shape, 1)
    wcol = jnp.sum(jnp.where(lane == e, wt_ref[...], 0.0), axis=1, keepdims=True)
    contrib = wcol * o

    @pl.when(e == 0)
    def _init():
        out_ref[...] = contrib

    @pl.when(e != 0)
    def _acc():
        out_ref[...] += contrib


@jax.jit
def kernel(x, gW, gb, W1, b1, W2, b2):
    del gb
    gw2 = gW.reshape(1, D)

    vals, idx, wts = pl.pallas_call(
        _gating_body,
        grid=(B // GATE_BB,),
        in_specs=[
            pl.BlockSpec((E, GATE_BB, D), lambda i: (0, i, 0)),
            pl.BlockSpec((1, D), lambda i: (0, 0)),
        ],
        out_specs=[
            pl.BlockSpec((GATE_BB, TOP_K), lambda i: (i, 0)),
            pl.BlockSpec((GATE_BB, TOP_K), lambda i: (i, 0)),
            pl.BlockSpec((GATE_BB, E), lambda i: (i, 0)),
        ],
        out_shape=[
            jax.ShapeDtypeStruct((B, TOP_K), jnp.float32),
            jax.ShapeDtypeStruct((B, TOP_K), jnp.int32),
            jax.ShapeDtypeStruct((B, E), jnp.float32),
        ],
    )(x, gw2)

    b1r = b1.reshape(E, 1, H)
    b2r = b2.reshape(E, 1, T)
    out = pl.pallas_call(
        _ffn_body,
        grid=(B // FFN_BB, E),
        in_specs=[
            pl.BlockSpec((1, FFN_BB, D), lambda i, e: (e, i, 0)),
            pl.BlockSpec((1, D, H), lambda i, e: (e, 0, 0)),
            pl.BlockSpec((1, 1, H), lambda i, e: (e, 0, 0)),
            pl.BlockSpec((1, H, T), lambda i, e: (e, 0, 0)),
            pl.BlockSpec((1, 1, T), lambda i, e: (e, 0, 0)),
            pl.BlockSpec((FFN_BB, E), lambda i, e: (i, 0)),
        ],
        out_specs=pl.BlockSpec((FFN_BB, T), lambda i, e: (i, 0)),
        out_shape=jax.ShapeDtypeStruct((B, T), jnp.float32),
        compiler_params=pltpu.CompilerParams(
            dimension_semantics=("arbitrary", "arbitrary"),
        ),
    )(x, W1, b1r, W2, b2r, wts)

    return (out, vals)
